# Initial kernel scaffold; baseline (speedup 1.0000x reference)
import jax
import jax.numpy as jnp
from jax.experimental import pallas as pl


def kernel(cat_idx, table):
    raise NotImplementedError("write your pallas kernel here")


# speedup vs baseline: 1.5752x; 1.5752x over previous
import functools

import jax
import jax.numpy as jnp
from jax import lax
from jax.experimental import pallas as pl
from jax.experimental.pallas import tpu as pltpu
from jax.experimental.pallas import tpu_sc as plsc

NUM_CATS = 1000000
EMBED_DIM = 32
BATCH = 16384
FIELDS = 26

_B = BATCH * FIELDS
_NC = 2
_NS = 16
_NW = _NC * _NS
_BPW = _B // _NW
_CHUNK = 1664
_NCHUNK = _BPW // _CHUNK


def _make_gather():
    mesh = plsc.VectorSubcoreMesh(core_axis_name="c", subcore_axis_name="s")

    @functools.partial(
        pl.kernel,
        mesh=mesh,
        compiler_params=pltpu.CompilerParams(use_tc_tiling_on_sc=False),
        out_type=jax.ShapeDtypeStruct((_B, EMBED_DIM), jnp.float32),
        scratch_types=[
            pltpu.VMEM((_BPW,), jnp.int32),
            pltpu.VMEM((_CHUNK, EMBED_DIM), jnp.float32),
            pltpu.VMEM((_CHUNK, EMBED_DIM), jnp.float32),
            pltpu.SemaphoreType.DMA,
            pltpu.SemaphoreType.DMA,
            pltpu.SemaphoreType.DMA,
            pltpu.SemaphoreType.DMA,
        ],
    )
    def gather_kernel(idx_hbm, table_hbm, out_hbm,
                      idx_v, rows0, rows1, g0, g1, s0, s1):
        wid = lax.axis_index("s") * _NC + lax.axis_index("c")
        base = wid * _BPW
        pltpu.sync_copy(idx_hbm.at[pl.ds(base, _BPW)], idx_v)

        rows = (rows0, rows1)
        gsem = (g0, g1)
        ssem = (s0, s1)
        gather = [None, None]
        store = [None, None]
        for c in range(_NCHUNK):
            cur = c % 2
            if c >= 2:
                store[cur].wait()
            gather[cur] = pltpu.async_copy(
                table_hbm.at[idx_v.at[pl.ds(c * _CHUNK, _CHUNK)]],
                rows[cur], gsem[cur])
            if c >= 1:
                prev = (c - 1) % 2
                gather[prev].wait()
                store[prev] = pltpu.async_copy(
                    rows[prev],
                    out_hbm.at[pl.ds(base + (c - 1) * _CHUNK, _CHUNK)],
                    ssem[prev])
        last = (_NCHUNK - 1) % 2
        gather[last].wait()
        store[last] = pltpu.async_copy(
            rows[last],
            out_hbm.at[pl.ds(base + (_NCHUNK - 1) * _CHUNK, _CHUNK)],
            ssem[last])
        store[(_NCHUNK - 2) % 2].wait()
        store[last].wait()

    return gather_kernel


_gather = _make_gather()


@jax.jit
def kernel(cat_idx, table):
    flat_idx = cat_idx.reshape(-1).astype(jnp.int32)
    out = _gather(flat_idx, table)
    return out.reshape(BATCH, FIELDS, EMBED_DIM)
